# baseline (speedup 1.0000x reference)
import functools

import jax
import jax.numpy as jnp
from jax import lax
from jax.experimental import pallas as pl
from jax.experimental.pallas import tpu as pltpu
from jax.experimental.pallas import tpu_sc as plsc

_B, _H, _W = 16, 512, 512
_N = _B * _H * _W
_NW = 32
_PER_W = _N // _NW
_CH = 16384
_NCH = _PER_W // _CH
_VPC = _CH // 16

_mesh = plsc.VectorSubcoreMesh(core_axis_name="c", subcore_axis_name="s")


@functools.partial(
    pl.kernel,
    out_type=[
        jax.ShapeDtypeStruct((_N,), jnp.float32),
        jax.ShapeDtypeStruct((_N,), jnp.float32),
    ],
    mesh=_mesh,
    scratch_types=[
        pltpu.VMEM((_CH,), jnp.float32),
        pltpu.VMEM((_CH,), jnp.float32),
        pltpu.VMEM((_CH,), jnp.float32),
        pltpu.VMEM((48,), jnp.float32),
        pltpu.VMEM((768,), jnp.float32),
        pltpu.VMEM_SHARED((768,), jnp.float32),
    ],
)
def _wsm_sc(x_hbm, o_ir_hbm, o_vis_hbm, in_v, o1_v, o2_v, part_v, comb_v, shared):
    c = lax.axis_index("c")
    s = lax.axis_index("s")
    wid = c * 16 + s
    base = wid * _PER_W

    big = jnp.full((16,), 1e9, jnp.float32)
    mf = big
    mn = big
    mx = jnp.full((16,), -1e9, jnp.float32)

    for k in range(_NCH):
        pltpu.sync_copy(x_hbm.at[pl.ds(base + k * _CH, _CH)], in_v)

        def body_a(i, carry):
            cmf, cmn, cmx = carry
            v = in_v[pl.ds(i * 16, 16)]
            x = v * 255.0
            xf = x.astype(jnp.int32).astype(jnp.float32)
            fr = x - xf
            return (
                jnp.minimum(cmf, fr),
                jnp.minimum(cmn, x),
                jnp.maximum(cmx, x),
            )

        mf, mn, mx = lax.fori_loop(0, _VPC, body_a, (mf, mn, mx))

    part_v[pl.ds(0, 16)] = mf
    part_v[pl.ds(16, 16)] = mn
    part_v[pl.ds(32, 16)] = mx
    pltpu.sync_copy(part_v, shared.at[pl.ds(s * 48, 48)])
    plsc.subcore_barrier()
    pltpu.sync_copy(shared, comb_v)

    sp = s ^ 1
    mf2 = jnp.minimum(comb_v[pl.ds(s * 48, 16)], comb_v[pl.ds(sp * 48, 16)])
    mn2 = jnp.minimum(
        comb_v[pl.ds(s * 48 + 16, 16)], comb_v[pl.ds(sp * 48 + 16, 16)]
    )
    mx2 = jnp.maximum(
        comb_v[pl.ds(s * 48 + 32, 16)], comb_v[pl.ds(sp * 48 + 32, 16)]
    )
    lanes = lax.iota(jnp.int32, 16)
    for sh in (8, 4, 2, 1):
        idx = (lanes + sh) & 15
        mf2 = jnp.minimum(mf2, mf2.at[idx].get(mode="promise_in_bounds"))
        mn2 = jnp.minimum(mn2, mn2.at[idx].get(mode="promise_in_bounds"))
        mx2 = jnp.maximum(mx2, mx2.at[idx].get(mode="promise_in_bounds"))
    any_exact = mf2 == 0.0
    bin_lo = (mn2 / 255.0 * 256.0).astype(jnp.int32)
    bin_hi = (mx2 / 255.0 * 256.0).astype(jnp.int32)
    flag = jnp.logical_and(any_exact, bin_lo != bin_hi)
    flag2 = jnp.where(flag, 2.0, 0.0)

    for k in range(_NCH):
        pltpu.sync_copy(x_hbm.at[pl.ds(base + k * _CH, _CH)], in_v)

        def body_b(i, carry):
            v = in_v[pl.ds(i * 16, 16)]
            t = v * flag2 - 1.0
            e = jnp.exp(-t)
            r = 1.0 / (1.0 + e)
            o1_v[pl.ds(i * 16, 16)] = r
            o2_v[pl.ds(i * 16, 16)] = 1.0 - r
            return carry

        lax.fori_loop(0, _VPC, body_b, 0)
        pltpu.sync_copy(o1_v, o_ir_hbm.at[pl.ds(base + k * _CH, _CH)])
        pltpu.sync_copy(o2_v, o_vis_hbm.at[pl.ds(base + k * _CH, _CH)])


def kernel(image_irr, image_vis):
    B, C, H, W = image_irr.shape
    x = image_irr.reshape(B * C * H * W)
    o_ir, o_vis = _wsm_sc(x)
    return (
        o_ir.reshape(B, C, H, W),
        o_vis.reshape(B, C, H, W),
    )


# speedup vs baseline: 8.6053x; 8.6053x over previous
import functools

import jax
import jax.numpy as jnp
from jax.experimental import pallas as pl
from jax.experimental.pallas import tpu as pltpu


def _wsm_kernel(x_ref, o_ir_ref, o_vis_ref):
    v = x_ref[...]
    x = v * 255.0
    frac = x - jnp.floor(x)
    any_exact = jnp.min(frac, axis=(1, 2), keepdims=True) == 0.0
    bin_lo = jnp.floor(jnp.min(x, axis=(1, 2), keepdims=True) / 255.0 * 256.0)
    bin_hi = jnp.floor(jnp.max(x, axis=(1, 2), keepdims=True) / 255.0 * 256.0)
    flag = jnp.logical_and(any_exact, bin_lo != bin_hi)

    t = jnp.where(flag, v * 2.0 - 1.0, -1.0)
    e = jnp.exp2(t * (-1.4426950408889634))
    r = 1.0 / (1.0 + e)
    o_ir_ref[...] = r
    o_vis_ref[...] = 1.0 - r


@functools.partial(jax.jit, static_argnames=())
def _run(x):
    B, H, W = x.shape
    NB = 4
    spec = pl.BlockSpec((NB, H, W), lambda b: (b, 0, 0))
    o_ir, o_vis = pl.pallas_call(
        _wsm_kernel,
        grid=(B // NB,),
        in_specs=[spec],
        out_specs=[spec, spec],
        out_shape=[
            jax.ShapeDtypeStruct((B, H, W), jnp.float32),
            jax.ShapeDtypeStruct((B, H, W), jnp.float32),
        ],
        compiler_params=pltpu.CompilerParams(
            dimension_semantics=("parallel",),
        ),
    )(x)
    return o_ir, o_vis


def kernel(image_irr, image_vis):
    B, C, H, W = image_irr.shape
    x = image_irr.reshape(B * C, H, W)
    o_ir, o_vis = _run(x)
    return (
        o_ir.reshape(B, C, H, W),
        o_vis.reshape(B, C, H, W),
    )
